# baseline (speedup 1.0000x reference)
import functools

import jax
import jax.numpy as jnp
from jax import lax
from jax.experimental import pallas as pl
from jax.experimental.pallas import tpu as pltpu
from jax.experimental.pallas import tpu_sc as plsc

N = 10000
E = 320000
D = 128

NC = 2
NS = 16
NW = NC * NS
L = 16

C = 80
PER_W = E // NW
N_CHUNKS = 128
PER_W_PAD = N_CHUNKS * C
NP1 = 10240
ZD1 = NP1 // NS
NP2 = 10112
ZD = NP2 // NS

_mesh = plsc.VectorSubcoreMesh(core_axis_name="c", subcore_axis_name="s")


@functools.partial(
    pl.kernel,
    mesh=_mesh,
    out_type=jax.ShapeDtypeStruct((NC, NP1), jnp.float32),
    scratch_types=[
        pltpu.VMEM((C,), jnp.int32),
        pltpu.VMEM((C,), jnp.float32),
        pltpu.VMEM((ZD1,), jnp.float32),
        pltpu.VMEM_SHARED((NP1,), jnp.float32),
    ],
)
def _sc_deg(dst_hbm, w_hbm, out_hbm, dst_v, w_v, zeros_v, dacc):
    cid = lax.axis_index("c")
    sid = lax.axis_index("s")
    wid = sid * NC + cid

    def _z(i, carry):
        zeros_v[pl.ds(i * L, L)] = jnp.zeros((L,), jnp.float32)
        return carry
    lax.fori_loop(0, ZD1 // L, _z, 0)
    pltpu.sync_copy(zeros_v, dacc.at[pl.ds(sid * ZD1, ZD1)])
    plsc.subcore_barrier()

    def chunk(j, carry):
        base = wid * PER_W_PAD + j * C
        pltpu.sync_copy(dst_hbm.at[pl.ds(base, C)], dst_v)
        pltpu.sync_copy(w_hbm.at[pl.ds(base, C)], w_v)
        pltpu.sync_copy(w_v, dacc.at[dst_v], add=True)
        return carry
    lax.fori_loop(0, N_CHUNKS, chunk, 0)

    plsc.subcore_barrier()
    pltpu.sync_copy(dacc.at[pl.ds(sid * ZD1, ZD1)],
                    out_hbm.at[cid, pl.ds(sid * ZD1, ZD1)])


@functools.partial(
    pl.kernel,
    mesh=_mesh,
    out_type=jax.ShapeDtypeStruct((NC, NP2, D), jnp.float32),
    scratch_types=(
        [pltpu.VMEM((C,), jnp.int32)] * 8
        + [pltpu.VMEM((C,), jnp.float32)] * 4
        + [pltpu.VMEM((C, D), jnp.float32)] * 4
        + [pltpu.VMEM_SHARED((NP2, D), jnp.float32)]
        + [pltpu.SemaphoreType.DMA] * 16
    ),
)
def _sc_scatter(y_hbm, src_hbm, dst_hbm, w_hbm, out_hbm, *bufargs):
    srcs = bufargs[0:4]
    dsts = bufargs[4:8]
    ws = bufargs[8:12]
    rfs = bufargs[12:16]
    acc = bufargs[16]
    gss = bufargs[17:21]
    sss = bufargs[21:25]
    iss = bufargs[25:29]
    iws = bufargs[29:33]

    cid = lax.axis_index("c")
    sid = lax.axis_index("s")
    wid = sid * NC + cid
    base_w = wid * PER_W_PAD

    def _zrow(i, carry):
        def _zcol(k, c2):
            rfs[0][i, pl.ds(k * L, L)] = jnp.zeros((L,), jnp.float32)
            return c2
        return lax.fori_loop(0, D // L, _zcol, carry)
    lax.fori_loop(0, C, _zrow, 0)
    for r in range(ZD // 8):
        pltpu.sync_copy(rfs[0].at[pl.ds(0, 8)],
                        acc.at[pl.ds(sid * ZD + r * 8, 8)])
    plsc.subcore_barrier()

    def scale(rf, wv):
        def blk(e16, c2):
            wreg = wv[pl.ds(e16 * L, L)]
            for lane in range(L):
                wb = jnp.take(wreg, jnp.full((L,), lane, jnp.int32))
                e = e16 * L + lane
                for k in range(D // L):
                    sl = pl.ds(k * L, L)
                    rf[e, sl] = rf[e, sl] * wb
            return c2
        lax.fori_loop(0, C // L, blk, 0)

    def cyc(j, p, q, ss_wait, pre2, pre4):
        pltpu.make_async_copy(dst_hbm.at[pl.ds(0, C)], dsts[p], iws[p]).wait()
        pltpu.make_async_copy(w_hbm.at[pl.ds(0, C)], ws[p], iws[p]).wait()
        pltpu.make_async_copy(y_hbm.at[srcs[p]], rfs[p], gss[p]).wait()
        scale(rfs[p], ws[p])
        pltpu.async_copy(rfs[p], acc.at[dsts[p]], sss[p], add=True)
        if pre2:
            if ss_wait:
                pltpu.make_async_copy(rfs[q], acc.at[dsts[q]], sss[q]).wait()
            pltpu.make_async_copy(src_hbm.at[pl.ds(0, C)], srcs[q],
                                  iss[q]).wait()
            pltpu.async_copy(y_hbm.at[srcs[q]], rfs[q], gss[q])
            b2 = base_w + (j + 2) * C
            pltpu.async_copy(dst_hbm.at[pl.ds(b2, C)], dsts[q], iws[q])
            pltpu.async_copy(w_hbm.at[pl.ds(b2, C)], ws[q], iws[q])
        if pre4:
            pltpu.async_copy(src_hbm.at[pl.ds(base_w + (j + 4) * C, C)],
                             srcs[p], iss[p])

    for p in (0, 1):
        pltpu.async_copy(src_hbm.at[pl.ds(base_w + p * C, C)], srcs[p],
                         iss[p])
    for p in (2, 3):
        pltpu.async_copy(src_hbm.at[pl.ds(base_w + p * C, C)], srcs[p],
                         iss[p])
    for p in (0, 1):
        pltpu.make_async_copy(src_hbm.at[pl.ds(0, C)], srcs[p], iss[p]).wait()
        pltpu.async_copy(y_hbm.at[srcs[p]], rfs[p], gss[p])
        pltpu.async_copy(dst_hbm.at[pl.ds(base_w + p * C, C)], dsts[p],
                         iws[p])
        pltpu.async_copy(w_hbm.at[pl.ds(base_w + p * C, C)], ws[p], iws[p])

    cyc(0, 0, 2, False, True, True)
    cyc(1, 1, 3, False, True, True)
    cyc(2, 2, 0, True, True, True)
    cyc(3, 3, 1, True, True, True)

    def quad(i, carry):
        j0 = 4 * i
        cyc(j0 + 0, 0, 2, True, True, True)
        cyc(j0 + 1, 1, 3, True, True, True)
        cyc(j0 + 2, 2, 0, True, True, True)
        cyc(j0 + 3, 3, 1, True, True, True)
        return carry
    lax.fori_loop(1, N_CHUNKS // 4 - 1, quad, 0)

    jt = N_CHUNKS - 4
    cyc(jt + 0, 0, 2, True, True, False)
    cyc(jt + 1, 1, 3, True, True, False)
    cyc(jt + 2, 2, 0, False, False, False)
    cyc(jt + 3, 3, 1, False, False, False)

    for p in range(4):
        pltpu.make_async_copy(rfs[p], acc.at[dsts[p]], sss[p]).wait()

    plsc.subcore_barrier()
    pltpu.sync_copy(acc.at[pl.ds(sid * ZD, ZD)],
                    out_hbm.at[cid, pl.ds(sid * ZD, ZD)])


RB = 1000


def _dinv_of(deg2_blk):
    deg = deg2_blk[:, 0] + deg2_blk[:, 1] + 1.0
    return lax.rsqrt(deg)


def _tc_first(deg2_ref, x_ref, w1_ref, y1_ref):
    dinv = _dinv_of(deg2_ref[...])
    xw = jnp.dot(x_ref[...], w1_ref[...], preferred_element_type=jnp.float32)
    y1_ref[...] = xw * dinv[:, None]


def _tc_mid(deg2_ref, acc_ref, y1_ref, b1_ref, w2_ref, y2_ref):
    dinv = _dinv_of(deg2_ref[...])
    s = acc_ref[0] + acc_ref[1] + y1_ref[...]
    h = jnp.maximum(s * dinv[:, None] + b1_ref[...], 0.0)
    y2_ref[...] = jnp.dot(h, w2_ref[...],
                          preferred_element_type=jnp.float32) * dinv[:, None]


def _tc_last(deg2_ref, acc_ref, y2_ref, b2_ref, out_ref):
    dinv = _dinv_of(deg2_ref[...])
    s = acc_ref[0] + acc_ref[1] + y2_ref[...]
    out_ref[...] = s * dinv[:, None] + b2_ref[...]


def _deg_spec():
    return pl.BlockSpec((RB, NC), lambda i: (i, 0))


def _row_spec():
    return pl.BlockSpec((RB, D), lambda i: (i, 0))


def _acc_spec():
    return pl.BlockSpec((NC, RB, D), lambda i: (0, i, 0))


def _full_spec(shape):
    nd = len(shape)
    return pl.BlockSpec(shape, lambda i: (0,) * nd)


def kernel(x, edge_index, edge_weight, W1, b1, W2, b2):
    src = edge_index[0].astype(jnp.int32)
    dst = edge_index[1].astype(jnp.int32)
    w = edge_weight.astype(jnp.float32)

    pad = PER_W_PAD - PER_W
    srcp = jnp.pad(src.reshape(NW, PER_W), ((0, 0), (0, pad))).reshape(-1)
    dstp = jnp.pad(dst.reshape(NW, PER_W), ((0, 0), (0, pad))).reshape(-1)
    wp = jnp.pad(w.reshape(NW, PER_W), ((0, 0), (0, pad))).reshape(-1)

    deg2 = _sc_deg(dstp, wp)[:, :N].T

    b1r = b1.reshape(1, D)
    b2r = b2.reshape(1, D)

    y1 = pl.pallas_call(
        _tc_first,
        grid=(N // RB,),
        in_specs=[_deg_spec(), _row_spec(), _full_spec((D, D))],
        out_specs=_row_spec(),
        out_shape=jax.ShapeDtypeStruct((N, D), jnp.float32),
    )(deg2, x, W1)

    acc1 = _sc_scatter(y1, srcp, dstp, wp)

    y2 = pl.pallas_call(
        _tc_mid,
        grid=(N // RB,),
        in_specs=[_deg_spec(), _acc_spec(), _row_spec(),
                  _full_spec((1, D)), _full_spec((D, D))],
        out_specs=_row_spec(),
        out_shape=jax.ShapeDtypeStruct((N, D), jnp.float32),
    )(deg2, acc1, y1, b1r, W2)

    acc2 = _sc_scatter(y2, srcp, dstp, wp)

    out = pl.pallas_call(
        _tc_last,
        grid=(N // RB,),
        in_specs=[_deg_spec(), _acc_spec(), _row_spec(), _full_spec((1, D))],
        out_specs=_row_spec(),
        out_shape=jax.ShapeDtypeStruct((N, D), jnp.float32),
    )(deg2, acc2, y2, b2r)

    return out


# speedup vs baseline: 1.7202x; 1.7202x over previous
import functools

import jax
import jax.numpy as jnp
from jax import lax
from jax.experimental import pallas as pl
from jax.experimental.pallas import tpu as pltpu
from jax.experimental.pallas import tpu_sc as plsc

N = 10000
E = 320000
D = 128

NC = 2
NS = 16
NW = NC * NS
L = 16

C = 112
PER_W = E // NW
N_CHUNKS = 90
PER_W_PAD = N_CHUNKS * C
NP1 = 10240
ZD1 = NP1 // NS
NP2 = 10112
ZD = NP2 // NS

_mesh = plsc.VectorSubcoreMesh(core_axis_name="c", subcore_axis_name="s")


@functools.partial(
    pl.kernel,
    mesh=_mesh,
    out_type=jax.ShapeDtypeStruct((NC, NP1), jnp.float32),
    scratch_types=[
        pltpu.VMEM((C,), jnp.int32),
        pltpu.VMEM((C,), jnp.float32),
        pltpu.VMEM((ZD1,), jnp.float32),
        pltpu.VMEM_SHARED((NP1,), jnp.float32),
    ],
)
def _sc_deg(dst_hbm, w_hbm, out_hbm, dst_v, w_v, zeros_v, dacc):
    cid = lax.axis_index("c")
    sid = lax.axis_index("s")
    wid = sid * NC + cid

    def _z(i, carry):
        zeros_v[pl.ds(i * L, L)] = jnp.zeros((L,), jnp.float32)
        return carry
    lax.fori_loop(0, ZD1 // L, _z, 0)
    pltpu.sync_copy(zeros_v, dacc.at[pl.ds(sid * ZD1, ZD1)])
    plsc.subcore_barrier()

    def chunk(j, carry):
        base = wid * PER_W_PAD + j * C
        pltpu.sync_copy(dst_hbm.at[pl.ds(base, C)], dst_v)
        pltpu.sync_copy(w_hbm.at[pl.ds(base, C)], w_v)
        pltpu.sync_copy(w_v, dacc.at[dst_v], add=True)
        return carry
    lax.fori_loop(0, N_CHUNKS, chunk, 0)

    plsc.subcore_barrier()
    pltpu.sync_copy(dacc.at[pl.ds(sid * ZD1, ZD1)],
                    out_hbm.at[cid, pl.ds(sid * ZD1, ZD1)])


@functools.partial(
    pl.kernel,
    mesh=_mesh,
    out_type=jax.ShapeDtypeStruct((NC, NP2, D), jnp.float32),
    scratch_types=(
        [pltpu.VMEM((C,), jnp.int32)] * 6
        + [pltpu.VMEM((C,), jnp.float32)] * 3
        + [pltpu.VMEM((C, D), jnp.float32)] * 3
        + [pltpu.VMEM_SHARED((NP2, D), jnp.float32)]
        + [pltpu.SemaphoreType.DMA] * 12
    ),
)
def _sc_scatter(y_hbm, src_hbm, dst_hbm, w_hbm, out_hbm, *bufargs):
    srcs = bufargs[0:3]
    dsts = bufargs[3:6]
    ws = bufargs[6:9]
    rfs = bufargs[9:12]
    acc = bufargs[12]
    gss = bufargs[13:16]
    sss = bufargs[16:19]
    iss = bufargs[19:22]
    iws = bufargs[22:25]

    cid = lax.axis_index("c")
    sid = lax.axis_index("s")
    wid = sid * NC + cid
    base_w = wid * PER_W_PAD

    def _zrow(i, carry):
        def _zcol(k, c2):
            rfs[0][i, pl.ds(k * L, L)] = jnp.zeros((L,), jnp.float32)
            return c2
        return lax.fori_loop(0, D // L, _zcol, carry)
    lax.fori_loop(0, C, _zrow, 0)
    for r in range(ZD // 8):
        pltpu.sync_copy(rfs[0].at[pl.ds(0, 8)],
                        acc.at[pl.ds(sid * ZD + r * 8, 8)])
    plsc.subcore_barrier()

    def scale(rf, wv):
        def blk(e16, c2):
            wreg = wv[pl.ds(e16 * L, L)]
            for lane in range(L):
                wb = jnp.take(wreg, jnp.full((L,), lane, jnp.int32))
                e = e16 * L + lane
                for k in range(D // L):
                    sl = pl.ds(k * L, L)
                    rf[e, sl] = rf[e, sl] * wb
            return c2
        lax.fori_loop(0, C // L, blk, 0)

    def cyc(j, p, q, ss_wait, pre2, pre3):
        pltpu.make_async_copy(dst_hbm.at[pl.ds(0, C)], dsts[p], iws[p]).wait()
        pltpu.make_async_copy(w_hbm.at[pl.ds(0, C)], ws[p], iws[p]).wait()
        pltpu.make_async_copy(y_hbm.at[srcs[p]], rfs[p], gss[p]).wait()
        scale(rfs[p], ws[p])
        pltpu.async_copy(rfs[p], acc.at[dsts[p]], sss[p], add=True)
        if pre2:
            if ss_wait:
                pltpu.make_async_copy(rfs[q], acc.at[dsts[q]], sss[q]).wait()
            pltpu.make_async_copy(src_hbm.at[pl.ds(0, C)], srcs[q],
                                  iss[q]).wait()
            pltpu.async_copy(y_hbm.at[srcs[q]], rfs[q], gss[q])
            b2 = base_w + (j + 2) * C
            pltpu.async_copy(dst_hbm.at[pl.ds(b2, C)], dsts[q], iws[q])
            pltpu.async_copy(w_hbm.at[pl.ds(b2, C)], ws[q], iws[q])
        if pre3:
            pltpu.async_copy(src_hbm.at[pl.ds(base_w + (j + 3) * C, C)],
                             srcs[p], iss[p])

    for p in (0, 1, 2):
        pltpu.async_copy(src_hbm.at[pl.ds(base_w + p * C, C)], srcs[p],
                         iss[p])
    for p in (0, 1):
        pltpu.make_async_copy(src_hbm.at[pl.ds(0, C)], srcs[p], iss[p]).wait()
        pltpu.async_copy(y_hbm.at[srcs[p]], rfs[p], gss[p])
        pltpu.async_copy(dst_hbm.at[pl.ds(base_w + p * C, C)], dsts[p],
                         iws[p])
        pltpu.async_copy(w_hbm.at[pl.ds(base_w + p * C, C)], ws[p], iws[p])

    cyc(0, 0, 2, False, True, True)
    cyc(1, 1, 0, True, True, True)
    cyc(2, 2, 1, True, True, True)

    def triple(i, carry):
        j0 = 3 * i
        cyc(j0 + 0, 0, 2, True, True, True)
        cyc(j0 + 1, 1, 0, True, True, True)
        cyc(j0 + 2, 2, 1, True, True, True)
        return carry
    lax.fori_loop(1, N_CHUNKS // 3 - 1, triple, 0)

    jt = N_CHUNKS - 3
    cyc(jt + 0, 0, 2, True, True, False)
    cyc(jt + 1, 1, 0, False, False, False)
    cyc(jt + 2, 2, 1, False, False, False)

    for p in range(3):
        pltpu.make_async_copy(rfs[p], acc.at[dsts[p]], sss[p]).wait()

    plsc.subcore_barrier()
    pltpu.sync_copy(acc.at[pl.ds(sid * ZD, ZD)],
                    out_hbm.at[cid, pl.ds(sid * ZD, ZD)])


RB = 1000


def _dinv_of(deg2_blk):
    deg = deg2_blk[:, 0] + deg2_blk[:, 1] + 1.0
    return lax.rsqrt(deg)


def _tc_first(deg2_ref, x_ref, w1_ref, y1_ref):
    dinv = _dinv_of(deg2_ref[...])
    xw = jnp.dot(x_ref[...], w1_ref[...], preferred_element_type=jnp.float32)
    y1_ref[...] = xw * dinv[:, None]


def _tc_mid(deg2_ref, acc_ref, y1_ref, b1_ref, w2_ref, y2_ref):
    dinv = _dinv_of(deg2_ref[...])
    s = acc_ref[0] + acc_ref[1] + y1_ref[...]
    h = jnp.maximum(s * dinv[:, None] + b1_ref[...], 0.0)
    y2_ref[...] = jnp.dot(h, w2_ref[...],
                          preferred_element_type=jnp.float32) * dinv[:, None]


def _tc_last(deg2_ref, acc_ref, y2_ref, b2_ref, out_ref):
    dinv = _dinv_of(deg2_ref[...])
    s = acc_ref[0] + acc_ref[1] + y2_ref[...]
    out_ref[...] = s * dinv[:, None] + b2_ref[...]


def _deg_spec():
    return pl.BlockSpec((RB, NC), lambda i: (i, 0))


def _row_spec():
    return pl.BlockSpec((RB, D), lambda i: (i, 0))


def _acc_spec():
    return pl.BlockSpec((NC, RB, D), lambda i: (0, i, 0))


def _full_spec(shape):
    nd = len(shape)
    return pl.BlockSpec(shape, lambda i: (0,) * nd)


def kernel(x, edge_index, edge_weight, W1, b1, W2, b2):
    src = edge_index[0].astype(jnp.int32)
    dst = edge_index[1].astype(jnp.int32)
    w = edge_weight.astype(jnp.float32)

    pad = PER_W_PAD - PER_W
    srcp = jnp.pad(src.reshape(NW, PER_W), ((0, 0), (0, pad))).reshape(-1)
    dstp = jnp.pad(dst.reshape(NW, PER_W), ((0, 0), (0, pad))).reshape(-1)
    wp = jnp.pad(w.reshape(NW, PER_W), ((0, 0), (0, pad))).reshape(-1)

    deg2 = _sc_deg(dstp, wp)[:, :N].T

    b1r = b1.reshape(1, D)
    b2r = b2.reshape(1, D)

    y1 = pl.pallas_call(
        _tc_first,
        grid=(N // RB,),
        in_specs=[_deg_spec(), _row_spec(), _full_spec((D, D))],
        out_specs=_row_spec(),
        out_shape=jax.ShapeDtypeStruct((N, D), jnp.float32),
    )(deg2, x, W1)

    acc1 = _sc_scatter(y1, srcp, dstp, wp)

    y2 = pl.pallas_call(
        _tc_mid,
        grid=(N // RB,),
        in_specs=[_deg_spec(), _acc_spec(), _row_spec(),
                  _full_spec((1, D)), _full_spec((D, D))],
        out_specs=_row_spec(),
        out_shape=jax.ShapeDtypeStruct((N, D), jnp.float32),
    )(deg2, acc1, y1, b1r, W2)

    acc2 = _sc_scatter(y2, srcp, dstp, wp)

    out = pl.pallas_call(
        _tc_last,
        grid=(N // RB,),
        in_specs=[_deg_spec(), _acc_spec(), _row_spec(), _full_spec((1, D))],
        out_specs=_row_spec(),
        out_shape=jax.ShapeDtypeStruct((N, D), jnp.float32),
    )(deg2, acc2, y2, b2r)

    return out


# speedup vs baseline: 1.9358x; 1.1253x over previous
import functools

import jax
import jax.numpy as jnp
from jax import lax
from jax.experimental import pallas as pl
from jax.experimental.pallas import tpu as pltpu
from jax.experimental.pallas import tpu_sc as plsc

N = 10000
E = 320000
D = 128

NC = 2
NS = 16
NW = NC * NS
L = 16

C = 112
PER_W = E // NW
N_CHUNKS = 90
PER_W_PAD = N_CHUNKS * C
NP1 = 10240
ZD1 = NP1 // NS
NP2 = 10112
ZD = NP2 // NS

_mesh = plsc.VectorSubcoreMesh(core_axis_name="c", subcore_axis_name="s")


@functools.partial(
    pl.kernel,
    mesh=_mesh,
    out_type=jax.ShapeDtypeStruct((NC, NP1), jnp.float32),
    scratch_types=(
        [pltpu.VMEM((C,), jnp.int32)] * 3
        + [pltpu.VMEM((C,), jnp.float32)] * 3
        + [pltpu.VMEM((ZD1,), jnp.float32)]
        + [pltpu.VMEM_SHARED((NP1,), jnp.float32)]
        + [pltpu.SemaphoreType.DMA] * 6
    ),
)
def _sc_deg(dst_hbm, w_hbm, out_hbm, *bufargs):
    dstv = bufargs[0:3]
    wv = bufargs[3:6]
    zeros_v = bufargs[6]
    dacc = bufargs[7]
    sss = bufargs[8:11]
    iws = bufargs[11:14]

    cid = lax.axis_index("c")
    sid = lax.axis_index("s")
    wid = sid * NC + cid
    base_w = wid * PER_W_PAD

    def _z(i, carry):
        zeros_v[pl.ds(i * L, L)] = jnp.zeros((L,), jnp.float32)
        return carry
    lax.fori_loop(0, ZD1 // L, _z, 0)
    pltpu.sync_copy(zeros_v, dacc.at[pl.ds(sid * ZD1, ZD1)])
    plsc.subcore_barrier()

    def cyc(j, p, q, ss_wait, pre2):
        pltpu.make_async_copy(dst_hbm.at[pl.ds(0, C)], dstv[p], iws[p]).wait()
        pltpu.make_async_copy(w_hbm.at[pl.ds(0, C)], wv[p], iws[p]).wait()
        pltpu.async_copy(wv[p], dacc.at[dstv[p]], sss[p], add=True)
        if pre2:
            if ss_wait:
                pltpu.make_async_copy(wv[q], dacc.at[dstv[q]], sss[q]).wait()
            b2 = base_w + (j + 2) * C
            pltpu.async_copy(dst_hbm.at[pl.ds(b2, C)], dstv[q], iws[q])
            pltpu.async_copy(w_hbm.at[pl.ds(b2, C)], wv[q], iws[q])

    for p in (0, 1):
        pltpu.async_copy(dst_hbm.at[pl.ds(base_w + p * C, C)], dstv[p],
                         iws[p])
        pltpu.async_copy(w_hbm.at[pl.ds(base_w + p * C, C)], wv[p], iws[p])

    cyc(0, 0, 2, False, True)
    cyc(1, 1, 0, True, True)
    cyc(2, 2, 1, True, True)

    def triple(i, carry):
        j0 = 3 * i
        cyc(j0 + 0, 0, 2, True, True)
        cyc(j0 + 1, 1, 0, True, True)
        cyc(j0 + 2, 2, 1, True, True)
        return carry
    lax.fori_loop(1, N_CHUNKS // 3 - 1, triple, 0)

    jt = N_CHUNKS - 3
    cyc(jt + 0, 0, 2, True, True)
    cyc(jt + 1, 1, 0, False, False)
    cyc(jt + 2, 2, 1, False, False)

    for p in range(3):
        pltpu.make_async_copy(wv[p], dacc.at[dstv[p]], sss[p]).wait()

    plsc.subcore_barrier()
    pltpu.sync_copy(dacc.at[pl.ds(sid * ZD1, ZD1)],
                    out_hbm.at[cid, pl.ds(sid * ZD1, ZD1)])


@functools.partial(
    pl.kernel,
    mesh=_mesh,
    out_type=jax.ShapeDtypeStruct((NC, NP2, D), jnp.float32),
    scratch_types=(
        [pltpu.VMEM((C,), jnp.int32)] * 6
        + [pltpu.VMEM((C,), jnp.float32)] * 3
        + [pltpu.VMEM((C, D), jnp.float32)] * 3
        + [pltpu.VMEM_SHARED((NP2, D), jnp.float32)]
        + [pltpu.SemaphoreType.DMA] * 12
    ),
)
def _sc_scatter(y_hbm, src_hbm, dst_hbm, w_hbm, out_hbm, *bufargs):
    srcs = bufargs[0:3]
    dsts = bufargs[3:6]
    ws = bufargs[6:9]
    rfs = bufargs[9:12]
    acc = bufargs[12]
    gss = bufargs[13:16]
    sss = bufargs[16:19]
    iss = bufargs[19:22]
    iws = bufargs[22:25]

    cid = lax.axis_index("c")
    sid = lax.axis_index("s")
    wid = sid * NC + cid
    base_w = wid * PER_W_PAD

    def _zrow(i, carry):
        def _zcol(k, c2):
            rfs[0][i, pl.ds(k * L, L)] = jnp.zeros((L,), jnp.float32)
            return c2
        return lax.fori_loop(0, D // L, _zcol, carry)
    lax.fori_loop(0, C, _zrow, 0)
    for r in range(ZD // 8):
        pltpu.sync_copy(rfs[0].at[pl.ds(0, 8)],
                        acc.at[pl.ds(sid * ZD + r * 8, 8)])
    plsc.subcore_barrier()

    def scale(rf, wv):
        def blk(e16, c2):
            wreg = wv[pl.ds(e16 * L, L)]
            for lane in range(L):
                wb = jnp.take(wreg, jnp.full((L,), lane, jnp.int32))
                e = e16 * L + lane
                for k in range(D // L):
                    sl = pl.ds(k * L, L)
                    rf[e, sl] = rf[e, sl] * wb
            return c2
        lax.fori_loop(0, C // L, blk, 0)

    def cyc(j, p, q, ss_wait, pre2, pre3):
        pltpu.make_async_copy(dst_hbm.at[pl.ds(0, C)], dsts[p], iws[p]).wait()
        pltpu.make_async_copy(w_hbm.at[pl.ds(0, C)], ws[p], iws[p]).wait()
        pltpu.make_async_copy(y_hbm.at[srcs[p]], rfs[p], gss[p]).wait()
        scale(rfs[p], ws[p])
        pltpu.async_copy(rfs[p], acc.at[dsts[p]], sss[p], add=True)
        if pre2:
            if ss_wait:
                pltpu.make_async_copy(rfs[q], acc.at[dsts[q]], sss[q]).wait()
            pltpu.make_async_copy(src_hbm.at[pl.ds(0, C)], srcs[q],
                                  iss[q]).wait()
            pltpu.async_copy(y_hbm.at[srcs[q]], rfs[q], gss[q])
            b2 = base_w + (j + 2) * C
            pltpu.async_copy(dst_hbm.at[pl.ds(b2, C)], dsts[q], iws[q])
            pltpu.async_copy(w_hbm.at[pl.ds(b2, C)], ws[q], iws[q])
        if pre3:
            pltpu.async_copy(src_hbm.at[pl.ds(base_w + (j + 3) * C, C)],
                             srcs[p], iss[p])

    for p in (0, 1, 2):
        pltpu.async_copy(src_hbm.at[pl.ds(base_w + p * C, C)], srcs[p],
                         iss[p])
    for p in (0, 1):
        pltpu.make_async_copy(src_hbm.at[pl.ds(0, C)], srcs[p], iss[p]).wait()
        pltpu.async_copy(y_hbm.at[srcs[p]], rfs[p], gss[p])
        pltpu.async_copy(dst_hbm.at[pl.ds(base_w + p * C, C)], dsts[p],
                         iws[p])
        pltpu.async_copy(w_hbm.at[pl.ds(base_w + p * C, C)], ws[p], iws[p])

    cyc(0, 0, 2, False, True, True)
    cyc(1, 1, 0, True, True, True)
    cyc(2, 2, 1, True, True, True)

    def triple(i, carry):
        j0 = 3 * i
        cyc(j0 + 0, 0, 2, True, True, True)
        cyc(j0 + 1, 1, 0, True, True, True)
        cyc(j0 + 2, 2, 1, True, True, True)
        return carry
    lax.fori_loop(1, N_CHUNKS // 3 - 1, triple, 0)

    jt = N_CHUNKS - 3
    cyc(jt + 0, 0, 2, True, True, False)
    cyc(jt + 1, 1, 0, False, False, False)
    cyc(jt + 2, 2, 1, False, False, False)

    for p in range(3):
        pltpu.make_async_copy(rfs[p], acc.at[dsts[p]], sss[p]).wait()

    plsc.subcore_barrier()
    pltpu.sync_copy(acc.at[pl.ds(sid * ZD, ZD)],
                    out_hbm.at[cid, pl.ds(sid * ZD, ZD)])


RB = 1000


def _dinv_of(deg2_blk):
    deg = deg2_blk[:, 0] + deg2_blk[:, 1] + 1.0
    return lax.rsqrt(deg)


def _tc_first(deg2_ref, x_ref, w1_ref, y1_ref):
    dinv = _dinv_of(deg2_ref[...])
    xw = jnp.dot(x_ref[...], w1_ref[...], preferred_element_type=jnp.float32)
    y1_ref[...] = xw * dinv[:, None]


def _tc_mid(deg2_ref, acc_ref, y1_ref, b1_ref, w2_ref, y2_ref):
    dinv = _dinv_of(deg2_ref[...])
    s = acc_ref[0] + acc_ref[1] + y1_ref[...]
    h = jnp.maximum(s * dinv[:, None] + b1_ref[...], 0.0)
    y2_ref[...] = jnp.dot(h, w2_ref[...],
                          preferred_element_type=jnp.float32) * dinv[:, None]


def _tc_last(deg2_ref, acc_ref, y2_ref, b2_ref, out_ref):
    dinv = _dinv_of(deg2_ref[...])
    s = acc_ref[0] + acc_ref[1] + y2_ref[...]
    out_ref[...] = s * dinv[:, None] + b2_ref[...]


def _deg_spec():
    return pl.BlockSpec((RB, NC), lambda i: (i, 0))


def _row_spec():
    return pl.BlockSpec((RB, D), lambda i: (i, 0))


def _acc_spec():
    return pl.BlockSpec((NC, RB, D), lambda i: (0, i, 0))


def _full_spec(shape):
    nd = len(shape)
    return pl.BlockSpec(shape, lambda i: (0,) * nd)


def kernel(x, edge_index, edge_weight, W1, b1, W2, b2):
    src = edge_index[0].astype(jnp.int32)
    dst = edge_index[1].astype(jnp.int32)
    w = edge_weight.astype(jnp.float32)

    pad = PER_W_PAD - PER_W
    srcp = jnp.pad(src.reshape(NW, PER_W), ((0, 0), (0, pad))).reshape(-1)
    dstp = jnp.pad(dst.reshape(NW, PER_W), ((0, 0), (0, pad))).reshape(-1)
    wp = jnp.pad(w.reshape(NW, PER_W), ((0, 0), (0, pad))).reshape(-1)

    deg2 = _sc_deg(dstp, wp)[:, :N].T

    b1r = b1.reshape(1, D)
    b2r = b2.reshape(1, D)

    y1 = pl.pallas_call(
        _tc_first,
        grid=(N // RB,),
        in_specs=[_deg_spec(), _row_spec(), _full_spec((D, D))],
        out_specs=_row_spec(),
        out_shape=jax.ShapeDtypeStruct((N, D), jnp.float32),
    )(deg2, x, W1)

    acc1 = _sc_scatter(y1, srcp, dstp, wp)

    y2 = pl.pallas_call(
        _tc_mid,
        grid=(N // RB,),
        in_specs=[_deg_spec(), _acc_spec(), _row_spec(),
                  _full_spec((1, D)), _full_spec((D, D))],
        out_specs=_row_spec(),
        out_shape=jax.ShapeDtypeStruct((N, D), jnp.float32),
    )(deg2, acc1, y1, b1r, W2)

    acc2 = _sc_scatter(y2, srcp, dstp, wp)

    out = pl.pallas_call(
        _tc_last,
        grid=(N // RB,),
        in_specs=[_deg_spec(), _acc_spec(), _row_spec(), _full_spec((1, D))],
        out_specs=_row_spec(),
        out_shape=jax.ShapeDtypeStruct((N, D), jnp.float32),
    )(deg2, acc2, y2, b2r)

    return out
